# baseline (speedup 1.0000x reference)
import functools

import jax
import jax.numpy as jnp
from jax import lax
from jax.experimental import pallas as pl
from jax.experimental.pallas import tpu as pltpu
from jax.experimental.pallas import tpu_sc as plsc

DIM = 128
MAX_P = 8
PB = 2048
HB = PB // MAX_P
NC, NS = 2, 16
NW = NC * NS
SL = 16


def _mesh():
    return plsc.VectorSubcoreMesh(core_axis_name="c", subcore_axis_name="s")


_SC_PARAMS = pltpu.CompilerParams(use_tc_tiling_on_sc=False)


def _sc_gather(table, idx):
    N = idx.shape[0]
    D = table.shape[-1]
    per_w = N // NW
    CH = min(per_w, 512)
    nch = per_w // CH

    @functools.partial(
        pl.kernel,
        out_type=jax.ShapeDtypeStruct((N, D), jnp.float32),
        mesh=_mesh(),
        scratch_types=[
            pltpu.VMEM((CH,), jnp.int32),
            pltpu.VMEM((CH, D), jnp.float32),
            pltpu.SemaphoreType.DMA,
        ],
        compiler_params=_SC_PARAMS,
        name=f"sc_gather_{N}",
    )
    def k(table_h, idx_h, out_h, idx_v, buf, sem):
        wid = lax.axis_index("s") * NC + lax.axis_index("c")
        base = wid * per_w
        for c in range(nch):
            pltpu.sync_copy(idx_h.at[pl.ds(base + c * CH, CH)], idx_v)
            pltpu.async_copy(table_h.at[idx_v], buf, sem).wait()
            pltpu.sync_copy(buf, out_h.at[pl.ds(base + c * CH, CH)])

    return k(table, idx)


def _sc_scatter_set(node_ref, idx3, upd):
    N, D = upd.shape
    nch, CH = idx3.shape[1], idx3.shape[2]

    @functools.partial(
        pl.kernel,
        out_type=(),
        mesh=_mesh(),
        scratch_types=[
            pltpu.VMEM((nch, CH), jnp.int32),
            pltpu.VMEM((CH, D), jnp.float32),
            pltpu.SemaphoreType.DMA,
        ],
        compiler_params=_SC_PARAMS,
        name="sc_scatter_set",
    )
    def k(idx_h, upd_h, node_h, idx_v, buf, sem):
        wid = lax.axis_index("s") * NC + lax.axis_index("c")
        base = wid * nch * CH
        pltpu.sync_copy(idx_h.at[wid], idx_v)
        for c in range(nch):
            pltpu.async_copy(upd_h.at[pl.ds(base + c * CH, CH)], buf, sem).wait()
            pltpu.sync_copy(buf, node_h.at[idx_v.at[c]])

    k(idx3, upd, node_ref)


def _sc_scatter_add(msgs, idx3s, v_pad):
    E, D = msgs.shape
    nchs, CHS = idx3s.shape[1], idx3s.shape[2]
    nsl = D // SL
    spc = nsl // NC
    rpt = v_pad // NS
    zr = rpt // 8

    @functools.partial(
        pl.kernel,
        out_type=jax.ShapeDtypeStruct((v_pad, D), jnp.float32),
        mesh=_mesh(),
        scratch_types=[
            pltpu.VMEM((nchs, CHS), jnp.int32),
            pltpu.VMEM((CHS, SL), jnp.float32),
            pltpu.VMEM((zr, SL), jnp.float32),
            pltpu.VMEM_SHARED((v_pad, SL), jnp.float32),
            pltpu.SemaphoreType.DMA,
        ],
        compiler_params=_SC_PARAMS,
        name="sc_scatter_add",
    )
    def k(msgs_h, idx_h, agg_h, idx_v, mbuf, zbuf, spm, sem):
        cid = lax.axis_index("c")
        sid = lax.axis_index("s")
        pltpu.sync_copy(idx_h.at[sid], idx_v)

        @pl.loop(0, zr)
        def _(i):
            zbuf[i] = jnp.zeros((SL,), jnp.float32)

        for sl in range(spc):
            s = cid * spc + sl
            for j in range(8):
                pltpu.sync_copy(zbuf, spm.at[pl.ds(sid * rpt + j * zr, zr)])
            plsc.subcore_barrier()
            for c in range(nchs):
                pltpu.sync_copy(
                    msgs_h.at[pl.ds(sid * nchs * CHS + c * CHS, CHS),
                              pl.ds(s * SL, SL)],
                    mbuf)
                pltpu.sync_copy(mbuf, spm.at[idx_v.at[c]], add=True)
            plsc.subcore_barrier()
            pltpu.sync_copy(
                spm.at[pl.ds(sid * rpt, rpt)],
                agg_h.at[pl.ds(sid * rpt, rpt), pl.ds(s * SL, SL)])
            plsc.subcore_barrier()

    return k(msgs, idx3s)


def _layernorm(x, w, b):
    m = x.mean(-1, keepdims=True)
    v = ((x - m) ** 2).mean(-1, keepdims=True)
    return (x - m) / jnp.sqrt(v + 1e-5) * w + b


def _elu(x):
    return jnp.where(x > 0, x, jnp.exp(x) - 1.0)


def _pair_kernel(v_ref, r_ref, aux_ref, hold_ref,
                 W1c_ref, b1c_ref, W2s_ref, b2s_ref,
                 lne_ref, hout_ref):
    cat = jnp.concatenate([v_ref[...], r_ref[...]], axis=1)
    T = jnp.maximum(cat @ W1c_ref[...] + b1c_ref[...], 0.0)
    Tm = jnp.concatenate(
        [T[:, ro * DIM:(ro + 1) * DIM] * aux_ref[:, ro:ro + 1]
         for ro in range(3)], axis=1)
    msgs = Tm @ W2s_ref[...] + aux_ref[:, 0:3] @ b2s_ref[...]
    agg = msgs.reshape(PB // MAX_P, MAX_P, DIM).sum(axis=1) * (1.0 / MAX_P)
    h = hold_ref[...] + _elu(agg)
    hout_ref[...] = _layernorm(h, lne_ref[0:1, :], lne_ref[1:2, :])


def _pair_stage(v_prev, r_prev, aux, h_emb, pw, NH, E):
    nblk = E // PB
    return pl.pallas_call(
        _pair_kernel,
        grid=(nblk,),
        in_specs=[
            pl.BlockSpec((PB, DIM), lambda i: (i, 0)),
            pl.BlockSpec((PB, DIM), lambda i: (i, 0)),
            pl.BlockSpec((PB, 8), lambda i: (i, 0)),
            pl.BlockSpec((PB // MAX_P, DIM), lambda i: (i, 0)),
            pl.BlockSpec((2 * DIM, 3 * DIM), lambda i: (0, 0)),
            pl.BlockSpec((1, 3 * DIM), lambda i: (0, 0)),
            pl.BlockSpec((3 * DIM, DIM), lambda i: (0, 0)),
            pl.BlockSpec((3, DIM), lambda i: (0, 0)),
            pl.BlockSpec((2, DIM), lambda i: (0, 0)),
        ],
        out_specs=pl.BlockSpec((PB // MAX_P, DIM), lambda i: (i, 0)),
        out_shape=jax.ShapeDtypeStruct((NH, DIM), jnp.float32),
    )(v_prev, r_prev, aux, h_emb,
      pw['W1c'], pw['b1c'], pw['W2s'], pw['b2s'], pw['ln_e'])


def _tab_kernel(ER, RR,
                hn_ref,
                Wen_ref, wbe_ref, Pen1_ref, pbe1_ref, Pen2_ref, pbe2_ref,
                Wrn_ref, wbr_ref, Prn1_ref, pbr1_ref, Prn2_ref, pbr2_ref,
                etab_ref, rtab_ref):
    hn = hn_ref[...]
    for ro in range(ER):
        t = hn @ Wen_ref[ro] + wbe_ref[ro]
        t = t @ Pen1_ref[ro] + pbe1_ref[ro]
        t = jnp.maximum(t, 0.0)
        etab_ref[ro] = t @ Pen2_ref[ro] + pbe2_ref[ro]
    for ro in range(RR):
        t = hn @ Wrn_ref[ro] + wbr_ref[ro]
        t = t @ Prn1_ref[ro] + pbr1_ref[ro]
        t = jnp.maximum(t, 0.0)
        rtab_ref[ro] = t @ Prn2_ref[ro] + pbr2_ref[ro]


TB = 512


def _tab_stage(hn, pw, ER, RR, NH):
    full = lambda *shape: pl.BlockSpec(shape, lambda i: (0,) * len(shape))
    return pl.pallas_call(
        functools.partial(_tab_kernel, ER, RR),
        grid=(NH // TB,),
        in_specs=[
            pl.BlockSpec((TB, DIM), lambda i: (i, 0)),
            full(ER, DIM, DIM), full(ER, DIM), full(ER, DIM, DIM), full(ER, DIM),
            full(ER, DIM, DIM), full(ER, DIM),
            full(RR, DIM, DIM), full(RR, DIM), full(RR, DIM, DIM), full(RR, DIM),
            full(RR, DIM, DIM), full(RR, DIM),
        ],
        out_specs=(
            pl.BlockSpec((ER, TB, DIM), lambda i: (0, i, 0)),
            pl.BlockSpec((RR, TB, DIM), lambda i: (0, i, 0)),
        ),
        out_shape=(
            jax.ShapeDtypeStruct((ER, NH, DIM), jnp.float32),
            jax.ShapeDtypeStruct((RR, NH, DIM), jnp.float32),
        ),
    )(hn,
      pw['Wen_w'], pw['Wen_b'], pw['Pen_w1'], pw['Pen_b1'], pw['Pen_w2'], pw['Pen_b2'],
      pw['Wrn_w'], pw['Wrn_b'], pw['Prn_w1'], pw['Prn_b1'], pw['Prn_w2'], pw['Prn_b2'])


def _upd_kernel(col, node_ref, agg_ref, aux_ref, ln_ref, out_ref):
    cnt = aux_ref[:, col:col + 1]
    x = node_ref[...] + _elu(agg_ref[...] / cnt)
    out_ref[...] = _layernorm(x, ln_ref[0:1, :], ln_ref[1:2, :])


def _upd_stage(node_pair, agg_pair, aux, ln, col, E):
    nblk = E // PB
    return pl.pallas_call(
        functools.partial(_upd_kernel, col),
        grid=(nblk,),
        in_specs=[
            pl.BlockSpec((PB, DIM), lambda i: (i, 0)),
            pl.BlockSpec((PB, DIM), lambda i: (i, 0)),
            pl.BlockSpec((PB, 8), lambda i: (i, 0)),
            pl.BlockSpec((2, DIM), lambda i: (0, 0)),
        ],
        out_specs=pl.BlockSpec((PB, DIM), lambda i: (i, 0)),
        out_shape=jax.ShapeDtypeStruct((E, DIM), jnp.float32),
    )(node_pair, agg_pair, aux, ln)


def kernel(node_emb, input_ids, fact_rel_ids, fact_ent_ids, fact_entity_roles,
           fact_rel_roles, fact_pair_mask, params):
    V = node_emb.shape[0]
    Bb, Hh, Pp = fact_ent_ids.shape
    E = Bb * Hh * Pp
    NH = Bb * Hh
    NUM_LAYERS, ER = params['Wen_w'].shape[:2]
    RR = params['Wrn_w'].shape[1]
    v_pad = ((V + NS * 8 - 1) // (NS * 8)) * NS * 8

    ent = fact_ent_ids.reshape(-1).astype(jnp.int32)
    rel = fact_rel_ids.reshape(-1).astype(jnp.int32)
    er = fact_entity_roles.reshape(-1).astype(jnp.int32)
    rr = fact_rel_roles.reshape(-1).astype(jnp.int32)

    counts_v = jnp.maximum(jnp.bincount(ent, length=V), 1).astype(jnp.float32)
    counts_r = jnp.maximum(jnp.bincount(rel, length=V), 1).astype(jnp.float32)
    aux = jnp.stack([
        (er == 0).astype(jnp.float32),
        (er == 1).astype(jnp.float32),
        (er == 2).astype(jnp.float32),
        jnp.zeros((E,), jnp.float32),
        jnp.zeros((E,), jnp.float32),
        counts_v[ent],
        counts_r[rel],
        jnp.zeros((E,), jnp.float32),
    ], axis=1)
    vr_idx = jnp.concatenate([ent, rel])
    h_of_e = jnp.arange(E, dtype=jnp.int32) // Pp
    sel_e = er * NH + h_of_e
    sel_r = rr * NH + h_of_e

    chw = (E // NW) // ((E // NW + 511) // 512)
    ent3 = ent.reshape(NW, -1, chw)
    rel3 = rel.reshape(NW, -1, chw)
    chs = (E // NS) // ((E // NS + 511) // 512)
    ent3s = ent.reshape(NS, -1, chs)
    rel3s = rel.reshape(NS, -1, chs)

    node_ref = jax.new_ref(node_emb)
    h_emb = jnp.zeros((NH, DIM), jnp.float32)
    for l in range(NUM_LAYERS):
        pw = {k: params[k][l] for k in (
            'Wen_w', 'Wen_b', 'Pen_w1', 'Pen_b1', 'Pen_w2', 'Pen_b2',
            'Wrn_w', 'Wrn_b', 'Prn_w1', 'Prn_b1', 'Prn_w2', 'Prn_b2')}
        pw['W1c'] = jnp.moveaxis(params['pair_W1'][l], 0, 1).reshape(2 * DIM, ER * DIM)
        pw['b1c'] = params['pair_b1'][l].reshape(1, ER * DIM)
        pw['W2s'] = params['pair_W2'][l].reshape(ER * DIM, DIM)
        pw['b2s'] = params['pair_b2'][l]
        pw['ln_e'] = jnp.stack([params['ln_e_w'][l], params['ln_e_b'][l]])
        ln_v = jnp.stack([params['ln_v_w'][l], params['ln_v_b'][l]])
        ln_r = jnp.stack([params['ln_r_w'][l], params['ln_r_b'][l]])

        vr_prev = _sc_gather(node_ref, vr_idx)
        h_emb = _pair_stage(vr_prev[:E], vr_prev[E:], aux, h_emb, pw, NH, E)
        etab, rtab = _tab_stage(h_emb, pw, ER, RR, NH)

        msgs_ent = _sc_gather(etab.reshape(ER * NH, DIM), sel_e)
        agg_v = _sc_scatter_add(msgs_ent, ent3s, v_pad)
        upd_ent = _upd_stage(vr_prev[:E], _sc_gather(agg_v, ent), aux, ln_v, 5, E)
        _sc_scatter_set(node_ref, ent3, upd_ent)

        msgs_rel = _sc_gather(rtab.reshape(RR * NH, DIM), sel_r)
        agg_r = _sc_scatter_add(msgs_rel, rel3s, v_pad)
        node_pair_r = _sc_gather(node_ref, rel)
        upd_rel = _upd_stage(node_pair_r, _sc_gather(agg_r, rel), aux, ln_r, 6, E)
        _sc_scatter_set(node_ref, rel3, upd_rel)

    x_global = _sc_gather(node_ref, input_ids.reshape(-1).astype(jnp.int32))
    x_global = x_global.reshape(Bb, input_ids.shape[1], DIM)
    node_out = node_ref[...]
    return x_global, node_out, h_emb


# speedup vs baseline: 1.0659x; 1.0659x over previous
import functools

import jax
import jax.numpy as jnp
from jax import lax
from jax.experimental import pallas as pl
from jax.experimental.pallas import tpu as pltpu
from jax.experimental.pallas import tpu_sc as plsc

DIM = 128
MAX_P = 8
PB = 2048
HB = PB // MAX_P
NC, NS = 2, 16
NW = NC * NS
SL = 16


def _mesh():
    return plsc.VectorSubcoreMesh(core_axis_name="c", subcore_axis_name="s")


_SC_PARAMS = pltpu.CompilerParams(use_tc_tiling_on_sc=False)


def _sc_gather(table, idx):
    N = idx.shape[0]
    D = table.shape[-1]
    per_w = N // NW
    CH = min(per_w, 512)
    nch = per_w // CH

    @functools.partial(
        pl.kernel,
        out_type=jax.ShapeDtypeStruct((N, D), jnp.float32),
        mesh=_mesh(),
        scratch_types=[
            pltpu.VMEM((CH,), jnp.int32),
            pltpu.VMEM((CH, D), jnp.float32),
            pltpu.SemaphoreType.DMA,
        ],
        compiler_params=_SC_PARAMS,
        name=f"sc_gather_{N}",
    )
    def k(table_h, idx_h, out_h, idx_v, buf, sem):
        wid = lax.axis_index("s") * NC + lax.axis_index("c")
        base = wid * per_w
        for c in range(nch):
            pltpu.sync_copy(idx_h.at[pl.ds(base + c * CH, CH)], idx_v)
            pltpu.async_copy(table_h.at[idx_v], buf, sem).wait()
            pltpu.sync_copy(buf, out_h.at[pl.ds(base + c * CH, CH)])

    return k(table, idx)


def _sc_scatter_set(node_ref, idx3, upd):
    N, D = upd.shape
    nch, CH = idx3.shape[1], idx3.shape[2]

    @functools.partial(
        pl.kernel,
        out_type=(),
        mesh=_mesh(),
        scratch_types=[
            pltpu.VMEM((nch, CH), jnp.int32),
            pltpu.VMEM((CH, D), jnp.float32),
            pltpu.SemaphoreType.DMA,
        ],
        compiler_params=_SC_PARAMS,
        name="sc_scatter_set",
    )
    def k(idx_h, upd_h, node_h, idx_v, buf, sem):
        wid = lax.axis_index("s") * NC + lax.axis_index("c")
        base = wid * nch * CH
        pltpu.sync_copy(idx_h.at[wid], idx_v)
        for c in range(nch):
            pltpu.async_copy(upd_h.at[pl.ds(base + c * CH, CH)], buf, sem).wait()
            pltpu.sync_copy(buf, node_h.at[idx_v.at[c]])

    k(idx3, upd, node_ref)


def _sc_scatter_add(msgs, idx3s, v_pad):
    E, D = msgs.shape
    nchs, CHS = idx3s.shape[1], idx3s.shape[2]
    nsl = D // SL
    spc = nsl // NC
    rpt = v_pad // NS
    zr = rpt // 8

    @functools.partial(
        pl.kernel,
        out_type=jax.ShapeDtypeStruct((v_pad, D), jnp.float32),
        mesh=_mesh(),
        scratch_types=[
            pltpu.VMEM((nchs, CHS), jnp.int32),
            pltpu.VMEM((CHS, SL), jnp.float32),
            pltpu.VMEM((CHS, SL), jnp.float32),
            pltpu.VMEM((zr, SL), jnp.float32),
            pltpu.VMEM_SHARED((v_pad, SL), jnp.float32),
            pltpu.SemaphoreType.DMA,
            pltpu.SemaphoreType.DMA,
        ],
        compiler_params=_SC_PARAMS,
        name="sc_scatter_add",
    )
    def k(msgs_h, idx_h, agg_h, idx_v, mbuf0, mbuf1, zbuf, spm, sem0, sem1):
        mbufs = (mbuf0, mbuf1)
        sems = (sem0, sem1)
        cid = lax.axis_index("c")
        sid = lax.axis_index("s")
        pltpu.sync_copy(idx_h.at[sid], idx_v)

        @pl.loop(0, zr)
        def _(i):
            zbuf[i] = jnp.zeros((SL,), jnp.float32)

        for sl in range(spc):
            s = cid * spc + sl
            for j in range(8):
                pltpu.sync_copy(zbuf, spm.at[pl.ds(sid * rpt + j * zr, zr)])
            plsc.subcore_barrier()
            cps = [None] * nchs
            cps[0] = pltpu.async_copy(
                msgs_h.at[pl.ds(sid * nchs * CHS, CHS), pl.ds(s * SL, SL)],
                mbufs[0], sems[0])
            for c in range(nchs):
                if c + 1 < nchs:
                    cps[c + 1] = pltpu.async_copy(
                        msgs_h.at[pl.ds(sid * nchs * CHS + (c + 1) * CHS, CHS),
                                  pl.ds(s * SL, SL)],
                        mbufs[(c + 1) % 2], sems[(c + 1) % 2])
                cps[c].wait()
                pltpu.sync_copy(mbufs[c % 2], spm.at[idx_v.at[c]], add=True)
            plsc.subcore_barrier()
            pltpu.sync_copy(
                spm.at[pl.ds(sid * rpt, rpt)],
                agg_h.at[pl.ds(sid * rpt, rpt), pl.ds(s * SL, SL)])
            plsc.subcore_barrier()

    return k(msgs, idx3s)


def _layernorm(x, w, b):
    m = x.mean(-1, keepdims=True)
    v = ((x - m) ** 2).mean(-1, keepdims=True)
    return (x - m) / jnp.sqrt(v + 1e-5) * w + b


def _elu(x):
    return jnp.where(x > 0, x, jnp.exp(x) - 1.0)


def _pair_kernel(v_ref, r_ref, aux_ref, hold_ref,
                 W1c_ref, b1c_ref, W2s_ref, b2s_ref,
                 lne_ref, hout_ref):
    cat = jnp.concatenate([v_ref[...], r_ref[...]], axis=1)
    T = jnp.maximum(cat @ W1c_ref[...] + b1c_ref[...], 0.0)
    Tm = jnp.concatenate(
        [T[:, ro * DIM:(ro + 1) * DIM] * aux_ref[:, ro:ro + 1]
         for ro in range(3)], axis=1)
    msgs = Tm @ W2s_ref[...] + aux_ref[:, 0:3] @ b2s_ref[...]
    agg = msgs.reshape(PB // MAX_P, MAX_P, DIM).sum(axis=1) * (1.0 / MAX_P)
    h = hold_ref[...] + _elu(agg)
    hout_ref[...] = _layernorm(h, lne_ref[0:1, :], lne_ref[1:2, :])


def _pair_stage(v_prev, r_prev, aux, h_emb, pw, NH, E):
    nblk = E // PB
    return pl.pallas_call(
        _pair_kernel,
        grid=(nblk,),
        in_specs=[
            pl.BlockSpec((PB, DIM), lambda i: (i, 0)),
            pl.BlockSpec((PB, DIM), lambda i: (i, 0)),
            pl.BlockSpec((PB, 8), lambda i: (i, 0)),
            pl.BlockSpec((PB // MAX_P, DIM), lambda i: (i, 0)),
            pl.BlockSpec((2 * DIM, 3 * DIM), lambda i: (0, 0)),
            pl.BlockSpec((1, 3 * DIM), lambda i: (0, 0)),
            pl.BlockSpec((3 * DIM, DIM), lambda i: (0, 0)),
            pl.BlockSpec((3, DIM), lambda i: (0, 0)),
            pl.BlockSpec((2, DIM), lambda i: (0, 0)),
        ],
        out_specs=pl.BlockSpec((PB // MAX_P, DIM), lambda i: (i, 0)),
        out_shape=jax.ShapeDtypeStruct((NH, DIM), jnp.float32),
    )(v_prev, r_prev, aux, h_emb,
      pw['W1c'], pw['b1c'], pw['W2s'], pw['b2s'], pw['ln_e'])


def _tab_kernel(ER, RR,
                hn_ref,
                Wen_ref, wbe_ref, Pen1_ref, pbe1_ref, Pen2_ref, pbe2_ref,
                Wrn_ref, wbr_ref, Prn1_ref, pbr1_ref, Prn2_ref, pbr2_ref,
                etab_ref, rtab_ref):
    hn = hn_ref[...]
    for ro in range(ER):
        t = hn @ Wen_ref[ro] + wbe_ref[ro]
        t = t @ Pen1_ref[ro] + pbe1_ref[ro]
        t = jnp.maximum(t, 0.0)
        etab_ref[ro] = t @ Pen2_ref[ro] + pbe2_ref[ro]
    for ro in range(RR):
        t = hn @ Wrn_ref[ro] + wbr_ref[ro]
        t = t @ Prn1_ref[ro] + pbr1_ref[ro]
        t = jnp.maximum(t, 0.0)
        rtab_ref[ro] = t @ Prn2_ref[ro] + pbr2_ref[ro]


TB = 512


def _tab_stage(hn, pw, ER, RR, NH):
    full = lambda *shape: pl.BlockSpec(shape, lambda i: (0,) * len(shape))
    return pl.pallas_call(
        functools.partial(_tab_kernel, ER, RR),
        grid=(NH // TB,),
        in_specs=[
            pl.BlockSpec((TB, DIM), lambda i: (i, 0)),
            full(ER, DIM, DIM), full(ER, DIM), full(ER, DIM, DIM), full(ER, DIM),
            full(ER, DIM, DIM), full(ER, DIM),
            full(RR, DIM, DIM), full(RR, DIM), full(RR, DIM, DIM), full(RR, DIM),
            full(RR, DIM, DIM), full(RR, DIM),
        ],
        out_specs=(
            pl.BlockSpec((ER, TB, DIM), lambda i: (0, i, 0)),
            pl.BlockSpec((RR, TB, DIM), lambda i: (0, i, 0)),
        ),
        out_shape=(
            jax.ShapeDtypeStruct((ER, NH, DIM), jnp.float32),
            jax.ShapeDtypeStruct((RR, NH, DIM), jnp.float32),
        ),
    )(hn,
      pw['Wen_w'], pw['Wen_b'], pw['Pen_w1'], pw['Pen_b1'], pw['Pen_w2'], pw['Pen_b2'],
      pw['Wrn_w'], pw['Wrn_b'], pw['Prn_w1'], pw['Prn_b1'], pw['Prn_w2'], pw['Prn_b2'])


def _upd_kernel(col, node_ref, agg_ref, aux_ref, ln_ref, out_ref):
    cnt = aux_ref[:, col:col + 1]
    x = node_ref[...] + _elu(agg_ref[...] / cnt)
    out_ref[...] = _layernorm(x, ln_ref[0:1, :], ln_ref[1:2, :])


def _upd_stage(node_pair, agg_pair, aux, ln, col, E):
    nblk = E // PB
    return pl.pallas_call(
        functools.partial(_upd_kernel, col),
        grid=(nblk,),
        in_specs=[
            pl.BlockSpec((PB, DIM), lambda i: (i, 0)),
            pl.BlockSpec((PB, DIM), lambda i: (i, 0)),
            pl.BlockSpec((PB, 8), lambda i: (i, 0)),
            pl.BlockSpec((2, DIM), lambda i: (0, 0)),
        ],
        out_specs=pl.BlockSpec((PB, DIM), lambda i: (i, 0)),
        out_shape=jax.ShapeDtypeStruct((E, DIM), jnp.float32),
    )(node_pair, agg_pair, aux, ln)


def kernel(node_emb, input_ids, fact_rel_ids, fact_ent_ids, fact_entity_roles,
           fact_rel_roles, fact_pair_mask, params):
    V = node_emb.shape[0]
    Bb, Hh, Pp = fact_ent_ids.shape
    E = Bb * Hh * Pp
    NH = Bb * Hh
    NUM_LAYERS, ER = params['Wen_w'].shape[:2]
    RR = params['Wrn_w'].shape[1]
    v_pad = ((V + NS * 8 - 1) // (NS * 8)) * NS * 8

    ent = fact_ent_ids.reshape(-1).astype(jnp.int32)
    rel = fact_rel_ids.reshape(-1).astype(jnp.int32)
    er = fact_entity_roles.reshape(-1).astype(jnp.int32)
    rr = fact_rel_roles.reshape(-1).astype(jnp.int32)

    counts_v = jnp.maximum(jnp.bincount(ent, length=V), 1).astype(jnp.float32)
    counts_r = jnp.maximum(jnp.bincount(rel, length=V), 1).astype(jnp.float32)
    aux = jnp.stack([
        (er == 0).astype(jnp.float32),
        (er == 1).astype(jnp.float32),
        (er == 2).astype(jnp.float32),
        jnp.zeros((E,), jnp.float32),
        jnp.zeros((E,), jnp.float32),
        counts_v[ent],
        counts_r[rel],
        jnp.zeros((E,), jnp.float32),
    ], axis=1)
    h_of_e = jnp.arange(E, dtype=jnp.int32) // Pp
    sel_e = er * NH + h_of_e
    sel_r = rr * NH + h_of_e

    chw = (E // NW) // ((E // NW + 511) // 512)
    ent3 = ent.reshape(NW, -1, chw)
    rel3 = rel.reshape(NW, -1, chw)
    chs = (E // NS) // ((E // NS + 255) // 256)
    ent3s = ent.reshape(NS, -1, chs)
    rel3s = rel.reshape(NS, -1, chs)

    node_ref = jax.new_ref(node_emb)
    h_emb = jnp.zeros((NH, DIM), jnp.float32)
    for l in range(NUM_LAYERS):
        pw = {k: params[k][l] for k in (
            'Wen_w', 'Wen_b', 'Pen_w1', 'Pen_b1', 'Pen_w2', 'Pen_b2',
            'Wrn_w', 'Wrn_b', 'Prn_w1', 'Prn_b1', 'Prn_w2', 'Prn_b2')}
        pw['W1c'] = jnp.moveaxis(params['pair_W1'][l], 0, 1).reshape(2 * DIM, ER * DIM)
        pw['b1c'] = params['pair_b1'][l].reshape(1, ER * DIM)
        pw['W2s'] = params['pair_W2'][l].reshape(ER * DIM, DIM)
        pw['b2s'] = params['pair_b2'][l]
        pw['ln_e'] = jnp.stack([params['ln_e_w'][l], params['ln_e_b'][l]])
        ln_v = jnp.stack([params['ln_v_w'][l], params['ln_v_b'][l]])
        ln_r = jnp.stack([params['ln_r_w'][l], params['ln_r_b'][l]])

        v_prev = _sc_gather(node_ref, ent)
        r_prev = _sc_gather(node_ref, rel)
        h_emb = _pair_stage(v_prev, r_prev, aux, h_emb, pw, NH, E)
        etab, rtab = _tab_stage(h_emb, pw, ER, RR, NH)

        msgs_ent = _sc_gather(etab.reshape(ER * NH, DIM), sel_e)
        agg_v = _sc_scatter_add(msgs_ent, ent3s, v_pad)
        upd_ent = _upd_stage(v_prev, _sc_gather(agg_v, ent), aux, ln_v, 5, E)
        _sc_scatter_set(node_ref, ent3, upd_ent)

        msgs_rel = _sc_gather(rtab.reshape(RR * NH, DIM), sel_r)
        agg_r = _sc_scatter_add(msgs_rel, rel3s, v_pad)
        node_pair_r = _sc_gather(node_ref, rel)
        upd_rel = _upd_stage(node_pair_r, _sc_gather(agg_r, rel), aux, ln_r, 6, E)
        _sc_scatter_set(node_ref, rel3, upd_rel)

    x_global = _sc_gather(node_ref, input_ids.reshape(-1).astype(jnp.int32))
    x_global = x_global.reshape(Bb, input_ids.shape[1], DIM)
    node_out = node_ref[...]
    return x_global, node_out, h_emb
